# baseline (speedup 1.0000x reference)
import functools

import jax
import jax.numpy as jnp
from jax import lax
from jax.experimental import pallas as pl
from jax.experimental.pallas import tpu as pltpu
from jax.experimental.pallas import tpu_sc as plsc

NC = 2
NS = 16
L = 16
NW = NC * NS

DIM = 64
CHUNK = 128
VPR = DIM // L


def _sc_body(x_hbm, table_hbm, out_hbm, idx_v, rows_v, gsem, osem):
    wid = lax.axis_index("s") * NC + lax.axis_index("c")
    n_chunks = x_hbm.shape[1]
    rows_per_w = n_chunks * CHUNK
    base = wid * rows_per_w

    pltpu.sync_copy(x_hbm.at[wid], idx_v)

    def chunk_body(c, _):
        pltpu.async_copy(table_hbm.at[idx_v.at[c]], rows_v, gsem).wait()

        def row_body(r, _):
            for j in range(VPR):
                v = rows_v[r, pl.ds(j * L, L)]
                rows_v[r, pl.ds(j * L, L)] = v / (1.0 + jnp.exp(-v))
            return 0

        lax.fori_loop(0, CHUNK, row_body, 0, unroll=2)
        pltpu.sync_copy(rows_v, out_hbm.at[pl.ds(base + c * CHUNK, CHUNK)])
        return 0

    lax.fori_loop(0, n_chunks, chunk_body, 0)


@jax.jit
def kernel(x, emb_weight):
    batch, fields = x.shape
    dim = emb_weight.shape[1]
    n_rows = batch * fields
    assert n_rows % (NW * CHUNK) == 0 and dim == DIM
    n_chunks = n_rows // (NW * CHUNK)

    x_split = x.reshape(NW, n_chunks, CHUNK).astype(jnp.int32)

    mesh = plsc.VectorSubcoreMesh(
        core_axis_name="c", subcore_axis_name="s", num_cores=NC, num_subcores=NS
    )
    run = pl.kernel(
        _sc_body,
        out_type=jax.ShapeDtypeStruct((n_rows, dim), jnp.float32),
        mesh=mesh,
        scratch_types=[
            pltpu.VMEM((n_chunks, CHUNK), jnp.int32),
            pltpu.VMEM((CHUNK, dim), jnp.float32),
            pltpu.SemaphoreType.DMA,
            pltpu.SemaphoreType.DMA,
        ],
        compiler_params=pltpu.CompilerParams(use_tc_tiling_on_sc=False),
    )
    out = run(x_split, emb_weight)
    return out.reshape(batch, fields, dim)


# speedup vs baseline: 1.1409x; 1.1409x over previous
import jax
import jax.numpy as jnp
from jax import lax
from jax.experimental import pallas as pl
from jax.experimental.pallas import tpu as pltpu
from jax.experimental.pallas import tpu_sc as plsc

NC = 2
NS = 16
L = 16
NW = NC * NS

DIM = 64
CHUNK = 128
NBUF = 4
LOOKAHEAD = 2
VPR = DIM // L


def _swish_inplace(buf):
    def row_body(r, _):
        for j in range(VPR):
            v = buf[r, pl.ds(j * L, L)]
            buf[r, pl.ds(j * L, L)] = v / (1.0 + jnp.exp(-v))
        return 0

    lax.fori_loop(0, CHUNK, row_body, 0, unroll=2)


def _sc_body(x_hbm, table_hbm, out_hbm, idx_v, bufs, gsems, osems):
    wid = lax.axis_index("s") * NC + lax.axis_index("c")
    n_chunks = x_hbm.shape[1]
    rows_per_w = n_chunks * CHUNK
    base = wid * rows_per_w

    pltpu.sync_copy(x_hbm.at[wid], idx_v)

    def fire_gather(c, b):
        pltpu.async_copy(table_hbm.at[idx_v.at[c]], bufs[b], gsems[b])

    def wait_gather(b):
        pltpu.make_async_copy(table_hbm.at[idx_v.at[0]], bufs[b], gsems[b]).wait()

    def fire_scatter(c, b):
        pltpu.async_copy(bufs[b], out_hbm.at[pl.ds(base + c * CHUNK, CHUNK)], osems[b])

    def wait_scatter(b):
        pltpu.make_async_copy(
            bufs[b], out_hbm.at[pl.ds(base, CHUNK)], osems[b]
        ).wait()

    for c in range(LOOKAHEAD):
        fire_gather(c, c % NBUF)

    def group_body(g, _):
        for b in range(NBUF):
            c = g * NBUF + b
            wait_gather(b)
            _swish_inplace(bufs[b])
            fire_scatter(c, b)

            @pl.when(c >= LOOKAHEAD)
            def _():
                wait_scatter((b - LOOKAHEAD) % NBUF)

            @pl.when(c + LOOKAHEAD < n_chunks)
            def _():
                fire_gather(c + LOOKAHEAD, (b + LOOKAHEAD) % NBUF)

        return 0

    lax.fori_loop(0, n_chunks // NBUF, group_body, 0)

    for k in range(LOOKAHEAD):
        wait_scatter((n_chunks - LOOKAHEAD + k) % NBUF)


@jax.jit
def kernel(x, emb_weight):
    batch, fields = x.shape
    dim = emb_weight.shape[1]
    n_rows = batch * fields
    assert n_rows % (NW * CHUNK * NBUF) == 0 and dim == DIM
    n_chunks = n_rows // (NW * CHUNK)

    x_split = x.reshape(NW, n_chunks, CHUNK).astype(jnp.int32)

    mesh = plsc.VectorSubcoreMesh(
        core_axis_name="c", subcore_axis_name="s", num_cores=NC, num_subcores=NS
    )
    run = pl.kernel(
        _sc_body,
        out_type=jax.ShapeDtypeStruct((n_rows, dim), jnp.float32),
        mesh=mesh,
        scratch_types=[
            pltpu.VMEM((n_chunks, CHUNK), jnp.int32),
            [pltpu.VMEM((CHUNK, dim), jnp.float32) for _ in range(NBUF)],
            [pltpu.SemaphoreType.DMA for _ in range(NBUF)],
            [pltpu.SemaphoreType.DMA for _ in range(NBUF)],
        ],
        compiler_params=pltpu.CompilerParams(use_tc_tiling_on_sc=False),
    )
    out = run(x_split, emb_weight)
    return out.reshape(batch, fields, dim)
